# baseline (speedup 1.0000x reference)
import functools

import jax
import jax.numpy as jnp
from jax import lax
from jax.experimental import pallas as pl
from jax.experimental.pallas import tpu as pltpu
from jax.experimental.pallas import tpu_sc as plsc

N_PTS = 16384
N_SMP = 1024
KNN = 16
CH = 64
_BIG_I32 = 1 << 30


def _fps_body(xyz_ref, pts_ref, dist_ref):
    x = xyz_ref[0]
    y = xyz_ref[1]
    z = xyz_ref[2]
    lin = (lax.broadcasted_iota(jnp.int32, (128, 128), 0) * 128
           + lax.broadcasted_iota(jnp.int32, (128, 128), 1))
    dist_ref[...] = jnp.full((128, 128), 1e10, jnp.float32)

    def step(i, farthest):
        mask = lin == farthest
        zf = jnp.float32(0.0)
        ax = (0, 1)
        cx = jnp.sum(jnp.where(mask, x, zf), axis=ax, keepdims=True)
        cy = jnp.sum(jnp.where(mask, y, zf), axis=ax, keepdims=True)
        cz = jnp.sum(jnp.where(mask, z, zf), axis=ax, keepdims=True)
        pts_ref[0, i] = cx[0, 0]
        pts_ref[1, i] = cy[0, 0]
        pts_ref[2, i] = cz[0, 0]
        dx = x - cx
        dy = y - cy
        dz = z - cz
        d = dx * dx + dy * dy + dz * dz
        old = dist_ref[...]
        nd = jnp.where(d < old, d, old)
        dist_ref[...] = nd
        m = jnp.max(nd, axis=ax, keepdims=True)
        return jnp.min(jnp.where(nd == m, lin, _BIG_I32), axis=ax,
                       keepdims=True)

    lax.fori_loop(0, N_SMP, step, jnp.zeros((1, 1), jnp.int32))


def _fps(points):
    xyz = points.T.reshape(3, 128, 128)
    pts = pl.pallas_call(
        _fps_body,
        out_shape=jax.ShapeDtypeStruct((3, N_SMP), jnp.float32),
        out_specs=pl.BlockSpec(memory_space=pltpu.SMEM),
        scratch_shapes=[pltpu.VMEM((128, 128), jnp.float32)],
    )(xyz)
    return pts.T


def _knn_body(q_ref, p_ref, idx_ref, cnt_ref, d_ref):
    @pl.when(pl.program_id(0) == 0)
    def _():
        cnt_ref[...] = jnp.zeros_like(cnt_ref)

    qx = q_ref[:, 0:1]
    qy = q_ref[:, 1:2]
    qz = q_ref[:, 2:3]
    px = p_ref[0:1, :]
    py = p_ref[1:2, :]
    pz = p_ref[2:3, :]
    dx = qx - px
    dy = qy - py
    dz = qz - pz
    d0 = dx * dx + dy * dy + dz * dz
    d_ref[...] = d0
    inf = jnp.float32(jnp.inf)
    m = jnp.min(d0, axis=1, keepdims=True)
    for j in range(KNN):
        dcur = d_ref[...]
        col = lax.broadcasted_iota(jnp.int32, (128, N_PTS), 1)
        sel = jnp.min(jnp.where(dcur == m, col, _BIG_I32), axis=1,
                      keepdims=True)
        idx_ref[:, j:j + 1] = sel
        dnew = jnp.where(col == sel, inf, dcur)
        d_ref[...] = dnew
        if j + 1 < KNN:
            m = jnp.min(dnew, axis=1, keepdims=True)
    taken = (d_ref[...] == inf).astype(jnp.float32)
    cnt_ref[...] += jnp.sum(taken, axis=0, keepdims=True)


def _knn(sampled_pts, points_t):
    grid = N_SMP // 128
    knn_idx, counts = pl.pallas_call(
        _knn_body,
        grid=(grid,),
        in_specs=[
            pl.BlockSpec((128, 3), lambda i: (i, 0)),
            pl.BlockSpec((3, N_PTS), lambda i: (0, 0)),
        ],
        out_specs=[
            pl.BlockSpec((128, KNN), lambda i: (i, 0)),
            pl.BlockSpec((1, N_PTS), lambda i: (0, 0)),
        ],
        out_shape=[
            jax.ShapeDtypeStruct((N_SMP, KNN), jnp.int32),
            jax.ShapeDtypeStruct((1, N_PTS), jnp.float32),
        ],
        scratch_shapes=[
            pltpu.VMEM((128, N_PTS), jnp.float32),
        ],
    )(sampled_pts, points_t)
    return knn_idx, counts


def _mlp_body(f_ref, w1_ref, g1_ref, b1_ref, w2_ref, g2_ref, b2_ref,
              cnt_ref, out_ref):
    scale = jnp.float32(1.0 / N_PTS)
    eps = jnp.float32(1e-5)
    cnt = cnt_ref[...]
    dn = (((1,), (1,)), ((), ()))
    g = lax.dot_general(f_ref[...], w1_ref[...], dn,
                        preferred_element_type=jnp.float32)
    m1 = jnp.sum(g * cnt, axis=0, keepdims=True) * scale
    c1 = g - m1
    v1 = jnp.sum(c1 * c1 * cnt, axis=0, keepdims=True) * scale
    a = jnp.maximum(c1 / jnp.sqrt(v1 + eps) * g1_ref[...] + b1_ref[...], 0.0)
    b = lax.dot_general(a, w2_ref[...], dn,
                        preferred_element_type=jnp.float32)
    m2 = jnp.sum(b * cnt, axis=0, keepdims=True) * scale
    c2 = b - m2
    v2 = jnp.sum(c2 * c2 * cnt, axis=0, keepdims=True) * scale
    c_final = jnp.maximum(
        c2 / jnp.sqrt(v2 + eps) * g2_ref[...] + b2_ref[...], 0.0)
    out_ref[...] = jnp.concatenate(
        [c_final, jnp.zeros((N_PTS, CH), jnp.float32)], axis=1)


def _mlp(features, W1, gamma1, beta1, W2, gamma2, beta2, counts_col):
    return pl.pallas_call(
        _mlp_body,
        out_shape=jax.ShapeDtypeStruct((N_PTS, 2 * CH), jnp.float32),
    )(features, W1, gamma1.reshape(1, CH), beta1.reshape(1, CH),
      W2, gamma2.reshape(1, CH), beta2.reshape(1, CH), counts_col)


def _sc_gather_max(c_rows, idx_flat):
    num_cores, num_subcores = 2, 16
    nw = num_cores * num_subcores
    per_w = N_SMP * KNN // nw
    chunk = 128
    n_chunks = per_w // chunk
    smp_per_chunk = chunk // KNN
    mesh = plsc.VectorSubcoreMesh(core_axis_name="c", subcore_axis_name="s",
                                  num_cores=num_cores,
                                  num_subcores=num_subcores)

    width = 2 * CH

    @functools.partial(
        pl.kernel,
        out_type=jax.ShapeDtypeStruct((N_SMP, width), jnp.float32),
        mesh=mesh,
        scratch_types=[
            pltpu.VMEM((chunk,), jnp.int32),
            pltpu.VMEM((chunk, width), jnp.float32),
            pltpu.VMEM((smp_per_chunk, width), jnp.float32),
            pltpu.SemaphoreType.DMA,
        ],
    )
    def run(c_hbm, idx_hbm, out_hbm, idx_v, rows_v, res_v, sem):
        wid = lax.axis_index("s") * num_cores + lax.axis_index("c")
        base = wid * per_w

        def do_chunk(ch, carry):
            pltpu.sync_copy(idx_hbm.at[pl.ds(base + ch * chunk, chunk)],
                            idx_v)
            pltpu.async_copy(c_hbm.at[idx_v], rows_v, sem).wait()
            for s in range(smp_per_chunk):
                for l in range(width // 16):
                    acc = rows_v[s * KNN, pl.ds(l * 16, 16)]
                    for j in range(1, KNN):
                        acc = jnp.maximum(
                            acc, rows_v[s * KNN + j, pl.ds(l * 16, 16)])
                    res_v[s, pl.ds(l * 16, 16)] = acc
            row0 = wid * (per_w // KNN) + ch * smp_per_chunk
            pltpu.sync_copy(res_v,
                            out_hbm.at[pl.ds(row0, smp_per_chunk)])
            return carry

        lax.fori_loop(0, n_chunks, do_chunk, jnp.int32(0))

    return run(c_rows, idx_flat)


def kernel(points, features, W1, gamma1, beta1, W2, gamma2, beta2):
    sampled_points = _fps(points)
    knn_idx, counts = _knn(sampled_points, points.T)
    counts_col = counts.reshape(N_PTS, 1)
    c_rows = _mlp(features, W1, gamma1, beta1, W2, gamma2, beta2, counts_col)
    out_features = _sc_gather_max(c_rows, knn_idx.reshape(-1))[:, :CH]
    return (sampled_points, out_features)


# speedup vs baseline: 1.0159x; 1.0159x over previous
import functools

import jax
import jax.numpy as jnp
from jax import lax
from jax.experimental import pallas as pl
from jax.experimental.pallas import tpu as pltpu
from jax.experimental.pallas import tpu_sc as plsc

N_PTS = 16384
N_SMP = 1024
KNN = 16
CH = 64
_BIG_I32 = 1 << 30


def _fps_body(xyz_ref, pts_ref, dist_ref):
    x = xyz_ref[0]
    y = xyz_ref[1]
    z = xyz_ref[2]
    lin = (lax.broadcasted_iota(jnp.int32, (128, 128), 0) * 128
           + lax.broadcasted_iota(jnp.int32, (128, 128), 1))
    dist_ref[...] = jnp.full((128, 128), 1e10, jnp.float32)

    li = lax.broadcasted_iota(jnp.int32, (1, 128), 1)

    def step(i, f):
        r = f // 128
        c = f - r * 128
        zf = jnp.float32(0.0)
        ax = (0, 1)
        sel = li == c
        xrow = xyz_ref[0, pl.ds(r, 1), :]
        yrow = xyz_ref[1, pl.ds(r, 1), :]
        zrow = xyz_ref[2, pl.ds(r, 1), :]
        cx = jnp.sum(jnp.where(sel, xrow, zf), axis=ax, keepdims=True)
        cy = jnp.sum(jnp.where(sel, yrow, zf), axis=ax, keepdims=True)
        cz = jnp.sum(jnp.where(sel, zrow, zf), axis=ax, keepdims=True)
        pts_ref[0, i] = cx[0, 0]
        pts_ref[1, i] = cy[0, 0]
        pts_ref[2, i] = cz[0, 0]
        dx = x - cx
        dy = y - cy
        dz = z - cz
        d = dx * dx + dy * dy + dz * dz
        old = dist_ref[...]
        nd = jnp.where(d < old, d, old)
        dist_ref[...] = nd
        m = jnp.max(nd, axis=ax, keepdims=True)
        return jnp.min(jnp.where(nd == m, lin, _BIG_I32))

    lax.fori_loop(0, N_SMP, step, jnp.int32(0))


def _fps(points):
    xyz = points.T.reshape(3, 128, 128)
    pts = pl.pallas_call(
        _fps_body,
        out_shape=jax.ShapeDtypeStruct((3, N_SMP), jnp.float32),
        out_specs=pl.BlockSpec(memory_space=pltpu.SMEM),
        scratch_shapes=[pltpu.VMEM((128, 128), jnp.float32)],
    )(xyz)
    return pts.T


def _knn_body(q_ref, p_ref, idx_ref, cnt_ref, d_ref):
    @pl.when(pl.program_id(0) == 0)
    def _():
        cnt_ref[...] = jnp.zeros_like(cnt_ref)

    qx = q_ref[:, 0:1]
    qy = q_ref[:, 1:2]
    qz = q_ref[:, 2:3]
    px = p_ref[0:1, :]
    py = p_ref[1:2, :]
    pz = p_ref[2:3, :]
    dx = qx - px
    dy = qy - py
    dz = qz - pz
    d0 = dx * dx + dy * dy + dz * dz
    d_ref[...] = d0
    inf = jnp.float32(jnp.inf)
    m = jnp.min(d0, axis=1, keepdims=True)
    for j in range(KNN):
        dcur = d_ref[...]
        col = lax.broadcasted_iota(jnp.int32, (128, N_PTS), 1)
        sel = jnp.min(jnp.where(dcur == m, col, _BIG_I32), axis=1,
                      keepdims=True)
        idx_ref[:, j:j + 1] = sel
        dnew = jnp.where(col == sel, inf, dcur)
        d_ref[...] = dnew
        if j + 1 < KNN:
            m = jnp.min(dnew, axis=1, keepdims=True)
    taken = (d_ref[...] == inf).astype(jnp.float32)
    cnt_ref[...] += jnp.sum(taken, axis=0, keepdims=True)


def _knn(sampled_pts, points_t):
    grid = N_SMP // 128
    knn_idx, counts = pl.pallas_call(
        _knn_body,
        grid=(grid,),
        in_specs=[
            pl.BlockSpec((128, 3), lambda i: (i, 0)),
            pl.BlockSpec((3, N_PTS), lambda i: (0, 0)),
        ],
        out_specs=[
            pl.BlockSpec((128, KNN), lambda i: (i, 0)),
            pl.BlockSpec((1, N_PTS), lambda i: (0, 0)),
        ],
        out_shape=[
            jax.ShapeDtypeStruct((N_SMP, KNN), jnp.int32),
            jax.ShapeDtypeStruct((1, N_PTS), jnp.float32),
        ],
        scratch_shapes=[
            pltpu.VMEM((128, N_PTS), jnp.float32),
        ],
    )(sampled_pts, points_t)
    return knn_idx, counts


def _mlp_body(f_ref, w1_ref, g1_ref, b1_ref, w2_ref, g2_ref, b2_ref,
              cnt_ref, out_ref):
    scale = jnp.float32(1.0 / N_PTS)
    eps = jnp.float32(1e-5)
    cnt = cnt_ref[...]
    dn = (((1,), (1,)), ((), ()))
    g = lax.dot_general(f_ref[...], w1_ref[...], dn,
                        preferred_element_type=jnp.float32)
    m1 = jnp.sum(g * cnt, axis=0, keepdims=True) * scale
    c1 = g - m1
    v1 = jnp.sum(c1 * c1 * cnt, axis=0, keepdims=True) * scale
    a = jnp.maximum(c1 / jnp.sqrt(v1 + eps) * g1_ref[...] + b1_ref[...], 0.0)
    b = lax.dot_general(a, w2_ref[...], dn,
                        preferred_element_type=jnp.float32)
    m2 = jnp.sum(b * cnt, axis=0, keepdims=True) * scale
    c2 = b - m2
    v2 = jnp.sum(c2 * c2 * cnt, axis=0, keepdims=True) * scale
    c_final = jnp.maximum(
        c2 / jnp.sqrt(v2 + eps) * g2_ref[...] + b2_ref[...], 0.0)
    out_ref[...] = jnp.concatenate(
        [c_final, jnp.zeros((N_PTS, CH), jnp.float32)], axis=1)


def _mlp(features, W1, gamma1, beta1, W2, gamma2, beta2, counts_col):
    return pl.pallas_call(
        _mlp_body,
        out_shape=jax.ShapeDtypeStruct((N_PTS, 2 * CH), jnp.float32),
    )(features, W1, gamma1.reshape(1, CH), beta1.reshape(1, CH),
      W2, gamma2.reshape(1, CH), beta2.reshape(1, CH), counts_col)


def _sc_gather_max(c_rows, idx_flat):
    num_cores, num_subcores = 2, 16
    nw = num_cores * num_subcores
    per_w = N_SMP * KNN // nw
    chunk = 128
    n_chunks = per_w // chunk
    smp_per_chunk = chunk // KNN
    mesh = plsc.VectorSubcoreMesh(core_axis_name="c", subcore_axis_name="s",
                                  num_cores=num_cores,
                                  num_subcores=num_subcores)

    width = 2 * CH

    @functools.partial(
        pl.kernel,
        out_type=jax.ShapeDtypeStruct((N_SMP, width), jnp.float32),
        mesh=mesh,
        scratch_types=[
            pltpu.VMEM((chunk,), jnp.int32),
            pltpu.VMEM((chunk, width), jnp.float32),
            pltpu.VMEM((smp_per_chunk, width), jnp.float32),
            pltpu.SemaphoreType.DMA,
        ],
    )
    def run(c_hbm, idx_hbm, out_hbm, idx_v, rows_v, res_v, sem):
        wid = lax.axis_index("s") * num_cores + lax.axis_index("c")
        base = wid * per_w

        def do_chunk(ch, carry):
            pltpu.sync_copy(idx_hbm.at[pl.ds(base + ch * chunk, chunk)],
                            idx_v)
            pltpu.async_copy(c_hbm.at[idx_v], rows_v, sem).wait()
            for s in range(smp_per_chunk):
                for l in range(width // 16):
                    acc = rows_v[s * KNN, pl.ds(l * 16, 16)]
                    for j in range(1, KNN):
                        acc = jnp.maximum(
                            acc, rows_v[s * KNN + j, pl.ds(l * 16, 16)])
                    res_v[s, pl.ds(l * 16, 16)] = acc
            row0 = wid * (per_w // KNN) + ch * smp_per_chunk
            pltpu.sync_copy(res_v,
                            out_hbm.at[pl.ds(row0, smp_per_chunk)])
            return carry

        lax.fori_loop(0, n_chunks, do_chunk, jnp.int32(0))

    return run(c_rows, idx_flat)


def kernel(points, features, W1, gamma1, beta1, W2, gamma2, beta2):
    sampled_points = _fps(points)
    knn_idx, counts = _knn(sampled_points, points.T)
    counts_col = counts.reshape(N_PTS, 1)
    c_rows = _mlp(features, W1, gamma1, beta1, W2, gamma2, beta2, counts_col)
    out_features = _sc_gather_max(c_rows, knn_idx.reshape(-1))[:, :CH]
    return (sampled_points, out_features)


# speedup vs baseline: 1.0646x; 1.0479x over previous
import functools

import jax
import jax.numpy as jnp
from jax import lax
from jax.experimental import pallas as pl
from jax.experimental.pallas import tpu as pltpu
from jax.experimental.pallas import tpu_sc as plsc

N_PTS = 16384
N_SMP = 1024
KNN = 16
CH = 64
_BIG_I32 = 1 << 30


def _fps_body(xyz_ref, pts_ref, dist_ref):
    x = xyz_ref[0]
    y = xyz_ref[1]
    z = xyz_ref[2]
    lin = (lax.broadcasted_iota(jnp.int32, (128, 128), 0) * 128
           + lax.broadcasted_iota(jnp.int32, (128, 128), 1))
    dist_ref[...] = jnp.full((128, 128), 1e10, jnp.float32)

    li = lax.broadcasted_iota(jnp.int32, (1, 128), 1)

    def step(i, f):
        r = f // 128
        c = f - r * 128
        zf = jnp.float32(0.0)
        ax = (0, 1)
        sel = li == c
        xrow = xyz_ref[0, pl.ds(r, 1), :]
        yrow = xyz_ref[1, pl.ds(r, 1), :]
        zrow = xyz_ref[2, pl.ds(r, 1), :]
        cx = jnp.sum(jnp.where(sel, xrow, zf), axis=ax, keepdims=True)
        cy = jnp.sum(jnp.where(sel, yrow, zf), axis=ax, keepdims=True)
        cz = jnp.sum(jnp.where(sel, zrow, zf), axis=ax, keepdims=True)
        pts_ref[0, i] = cx[0, 0]
        pts_ref[1, i] = cy[0, 0]
        pts_ref[2, i] = cz[0, 0]
        dx = x - cx
        dy = y - cy
        dz = z - cz
        d = dx * dx + dy * dy + dz * dz
        old = dist_ref[...]
        nd = jnp.where(d < old, d, old)
        dist_ref[...] = nd
        m = jnp.max(nd, axis=ax, keepdims=True)
        return jnp.min(jnp.where(nd == m, lin, _BIG_I32))

    lax.fori_loop(0, N_SMP, step, jnp.int32(0))


def _fps(points):
    xyz = points.T.reshape(3, 128, 128)
    pts = pl.pallas_call(
        _fps_body,
        out_shape=jax.ShapeDtypeStruct((3, N_SMP), jnp.float32),
        out_specs=pl.BlockSpec(memory_space=pltpu.SMEM),
        scratch_shapes=[pltpu.VMEM((128, 128), jnp.float32)],
    )(xyz)
    return pts.T


def _knn_body(q_ref, p_ref, idx_ref, cnt_ref, d_ref):
    @pl.when(pl.program_id(0) == 0)
    def _():
        cnt_ref[...] = jnp.zeros_like(cnt_ref)

    qx = q_ref[:, 0:1]
    qy = q_ref[:, 1:2]
    qz = q_ref[:, 2:3]
    px = p_ref[0:1, :]
    py = p_ref[1:2, :]
    pz = p_ref[2:3, :]
    dx = qx - px
    dy = qy - py
    dz = qz - pz
    d0 = dx * dx + dy * dy + dz * dz
    d_ref[...] = d0
    inf = jnp.float32(jnp.inf)
    m = jnp.min(d0, axis=1, keepdims=True)
    for j in range(KNN):
        dcur = d_ref[...]
        col = lax.broadcasted_iota(jnp.int32, (d_ref.shape[0], N_PTS), 1)
        sel = jnp.min(jnp.where(dcur == m, col, _BIG_I32), axis=1,
                      keepdims=True)
        idx_ref[:, j:j + 1] = sel
        dnew = jnp.where(col == sel, inf, dcur)
        d_ref[...] = dnew
        if j + 1 < KNN:
            m = jnp.min(dnew, axis=1, keepdims=True)
    taken = (d_ref[...] == inf).astype(jnp.float32)
    cnt_ref[...] += jnp.sum(taken, axis=0, keepdims=True)


def _knn(sampled_pts, points_t):
    rows = 256
    grid = N_SMP // rows
    knn_idx, counts = pl.pallas_call(
        _knn_body,
        grid=(grid,),
        in_specs=[
            pl.BlockSpec((rows, 3), lambda i: (i, 0)),
            pl.BlockSpec((3, N_PTS), lambda i: (0, 0)),
        ],
        out_specs=[
            pl.BlockSpec((rows, KNN), lambda i: (i, 0)),
            pl.BlockSpec((1, N_PTS), lambda i: (0, 0)),
        ],
        out_shape=[
            jax.ShapeDtypeStruct((N_SMP, KNN), jnp.int32),
            jax.ShapeDtypeStruct((1, N_PTS), jnp.float32),
        ],
        scratch_shapes=[
            pltpu.VMEM((rows, N_PTS), jnp.float32),
        ],
    )(sampled_pts, points_t)
    return knn_idx, counts


def _mlp_body(f_ref, w1_ref, g1_ref, b1_ref, w2_ref, g2_ref, b2_ref,
              cnt_ref, out_ref):
    scale = jnp.float32(1.0 / N_PTS)
    eps = jnp.float32(1e-5)
    cnt = cnt_ref[...]
    dn = (((1,), (1,)), ((), ()))
    g = lax.dot_general(f_ref[...], w1_ref[...], dn,
                        preferred_element_type=jnp.float32)
    m1 = jnp.sum(g * cnt, axis=0, keepdims=True) * scale
    c1 = g - m1
    v1 = jnp.sum(c1 * c1 * cnt, axis=0, keepdims=True) * scale
    a = jnp.maximum(c1 / jnp.sqrt(v1 + eps) * g1_ref[...] + b1_ref[...], 0.0)
    b = lax.dot_general(a, w2_ref[...], dn,
                        preferred_element_type=jnp.float32)
    m2 = jnp.sum(b * cnt, axis=0, keepdims=True) * scale
    c2 = b - m2
    v2 = jnp.sum(c2 * c2 * cnt, axis=0, keepdims=True) * scale
    c_final = jnp.maximum(
        c2 / jnp.sqrt(v2 + eps) * g2_ref[...] + b2_ref[...], 0.0)
    out_ref[...] = jnp.concatenate(
        [c_final, jnp.zeros((N_PTS, CH), jnp.float32)], axis=1)


def _mlp(features, W1, gamma1, beta1, W2, gamma2, beta2, counts_col):
    return pl.pallas_call(
        _mlp_body,
        out_shape=jax.ShapeDtypeStruct((N_PTS, 2 * CH), jnp.float32),
    )(features, W1, gamma1.reshape(1, CH), beta1.reshape(1, CH),
      W2, gamma2.reshape(1, CH), beta2.reshape(1, CH), counts_col)


def _sc_gather_max(c_rows, idx_flat):
    num_cores, num_subcores = 2, 16
    nw = num_cores * num_subcores
    per_w = N_SMP * KNN // nw
    chunk = 128
    n_chunks = per_w // chunk
    smp_per_chunk = chunk // KNN
    mesh = plsc.VectorSubcoreMesh(core_axis_name="c", subcore_axis_name="s",
                                  num_cores=num_cores,
                                  num_subcores=num_subcores)

    width = 2 * CH

    @functools.partial(
        pl.kernel,
        out_type=jax.ShapeDtypeStruct((N_SMP, width), jnp.float32),
        mesh=mesh,
        scratch_types=[
            pltpu.VMEM((chunk,), jnp.int32),
            pltpu.VMEM((chunk, width), jnp.float32),
            pltpu.VMEM((smp_per_chunk, width), jnp.float32),
            pltpu.SemaphoreType.DMA,
        ],
    )
    def run(c_hbm, idx_hbm, out_hbm, idx_v, rows_v, res_v, sem):
        wid = lax.axis_index("s") * num_cores + lax.axis_index("c")
        base = wid * per_w

        def do_chunk(ch, carry):
            pltpu.sync_copy(idx_hbm.at[pl.ds(base + ch * chunk, chunk)],
                            idx_v)
            pltpu.async_copy(c_hbm.at[idx_v], rows_v, sem).wait()
            for s in range(smp_per_chunk):
                for l in range(width // 16):
                    acc = rows_v[s * KNN, pl.ds(l * 16, 16)]
                    for j in range(1, KNN):
                        acc = jnp.maximum(
                            acc, rows_v[s * KNN + j, pl.ds(l * 16, 16)])
                    res_v[s, pl.ds(l * 16, 16)] = acc
            row0 = wid * (per_w // KNN) + ch * smp_per_chunk
            pltpu.sync_copy(res_v,
                            out_hbm.at[pl.ds(row0, smp_per_chunk)])
            return carry

        lax.fori_loop(0, n_chunks, do_chunk, jnp.int32(0))

    return run(c_rows, idx_flat)


def kernel(points, features, W1, gamma1, beta1, W2, gamma2, beta2):
    sampled_points = _fps(points)
    knn_idx, counts = _knn(sampled_points, points.T)
    counts_col = counts.reshape(N_PTS, 1)
    c_rows = _mlp(features, W1, gamma1, beta1, W2, gamma2, beta2, counts_col)
    out_features = _sc_gather_max(c_rows, knn_idx.reshape(-1))[:, :CH]
    return (sampled_points, out_features)


# speedup vs baseline: 1.1018x; 1.0350x over previous
import functools

import jax
import jax.numpy as jnp
from jax import lax
from jax.experimental import pallas as pl
from jax.experimental.pallas import tpu as pltpu
from jax.experimental.pallas import tpu_sc as plsc

N_PTS = 16384
N_SMP = 1024
KNN = 16
CH = 64
_BIG_I32 = 1 << 30


def _fps_body(xyz_ref, pts_ref):
    x = xyz_ref[0]
    y = xyz_ref[1]
    z = xyz_ref[2]
    lin = (lax.broadcasted_iota(jnp.int32, (128, 128), 0) * 128
           + lax.broadcasted_iota(jnp.int32, (128, 128), 1))
    li = lax.broadcasted_iota(jnp.int32, (1, 128), 1)

    def step(i, carry):
        dist, f = carry
        r = f // 128
        c = f - r * 128
        zf = jnp.float32(0.0)
        ax = (0, 1)
        sel = li == c
        xrow = xyz_ref[0, pl.ds(r, 1), :]
        yrow = xyz_ref[1, pl.ds(r, 1), :]
        zrow = xyz_ref[2, pl.ds(r, 1), :]
        cx = jnp.sum(jnp.where(sel, xrow, zf), axis=ax, keepdims=True)
        cy = jnp.sum(jnp.where(sel, yrow, zf), axis=ax, keepdims=True)
        cz = jnp.sum(jnp.where(sel, zrow, zf), axis=ax, keepdims=True)
        pts_ref[0, i] = cx[0, 0]
        pts_ref[1, i] = cy[0, 0]
        pts_ref[2, i] = cz[0, 0]
        dx = x - cx
        dy = y - cy
        dz = z - cz
        d = dx * dx + dy * dy + dz * dz
        nd = jnp.where(d < dist, d, dist)
        m = jnp.max(nd, axis=ax, keepdims=True)
        return nd, jnp.min(jnp.where(nd == m, lin, _BIG_I32))

    lax.fori_loop(0, N_SMP, step,
                  (jnp.full((128, 128), 1e10, jnp.float32), jnp.int32(0)))


def _fps(points):
    xyz = points.T.reshape(3, 128, 128)
    pts = pl.pallas_call(
        _fps_body,
        out_shape=jax.ShapeDtypeStruct((3, N_SMP), jnp.float32),
        out_specs=pl.BlockSpec(memory_space=pltpu.SMEM),
    )(xyz)
    return pts.T


def _knn_body(q_ref, p_ref, idx_ref, cnt_ref, d_ref):
    @pl.when(pl.program_id(0) == 0)
    def _():
        cnt_ref[...] = jnp.zeros_like(cnt_ref)

    qx = q_ref[:, 0:1]
    qy = q_ref[:, 1:2]
    qz = q_ref[:, 2:3]
    px = p_ref[0:1, :]
    py = p_ref[1:2, :]
    pz = p_ref[2:3, :]
    dx = qx - px
    dy = qy - py
    dz = qz - pz
    d0 = dx * dx + dy * dy + dz * dz
    d_ref[...] = d0
    inf = jnp.float32(jnp.inf)
    m = jnp.min(d0, axis=1, keepdims=True)
    for j in range(KNN):
        dcur = d_ref[...]
        col = lax.broadcasted_iota(jnp.int32, (d_ref.shape[0], N_PTS), 1)
        sel = jnp.min(jnp.where(dcur == m, col, _BIG_I32), axis=1,
                      keepdims=True)
        idx_ref[:, j:j + 1] = sel
        dnew = jnp.where(col == sel, inf, dcur)
        d_ref[...] = dnew
        if j + 1 < KNN:
            m = jnp.min(dnew, axis=1, keepdims=True)
    taken = (d_ref[...] == inf).astype(jnp.float32)
    cnt_ref[...] += jnp.sum(taken, axis=0, keepdims=True)


def _knn(sampled_pts, points_t):
    rows = 512
    grid = N_SMP // rows
    knn_idx, counts = pl.pallas_call(
        _knn_body,
        grid=(grid,),
        in_specs=[
            pl.BlockSpec((rows, 3), lambda i: (i, 0)),
            pl.BlockSpec((3, N_PTS), lambda i: (0, 0)),
        ],
        out_specs=[
            pl.BlockSpec((rows, KNN), lambda i: (i, 0)),
            pl.BlockSpec((1, N_PTS), lambda i: (0, 0)),
        ],
        out_shape=[
            jax.ShapeDtypeStruct((N_SMP, KNN), jnp.int32),
            jax.ShapeDtypeStruct((1, N_PTS), jnp.float32),
        ],
        scratch_shapes=[
            pltpu.VMEM((rows, N_PTS), jnp.float32),
        ],
    )(sampled_pts, points_t)
    return knn_idx, counts


def _mlp_body(f_ref, w1_ref, g1_ref, b1_ref, w2_ref, g2_ref, b2_ref,
              cnt_ref, out_ref):
    scale = jnp.float32(1.0 / N_PTS)
    eps = jnp.float32(1e-5)
    cnt = cnt_ref[...]
    dn = (((1,), (1,)), ((), ()))
    g = lax.dot_general(f_ref[...], w1_ref[...], dn,
                        preferred_element_type=jnp.float32)
    m1 = jnp.sum(g * cnt, axis=0, keepdims=True) * scale
    c1 = g - m1
    v1 = jnp.sum(c1 * c1 * cnt, axis=0, keepdims=True) * scale
    a = jnp.maximum(c1 / jnp.sqrt(v1 + eps) * g1_ref[...] + b1_ref[...], 0.0)
    b = lax.dot_general(a, w2_ref[...], dn,
                        preferred_element_type=jnp.float32)
    m2 = jnp.sum(b * cnt, axis=0, keepdims=True) * scale
    c2 = b - m2
    v2 = jnp.sum(c2 * c2 * cnt, axis=0, keepdims=True) * scale
    c_final = jnp.maximum(
        c2 / jnp.sqrt(v2 + eps) * g2_ref[...] + b2_ref[...], 0.0)
    out_ref[...] = jnp.concatenate(
        [c_final, jnp.zeros((N_PTS, CH), jnp.float32)], axis=1)


def _mlp(features, W1, gamma1, beta1, W2, gamma2, beta2, counts_col):
    return pl.pallas_call(
        _mlp_body,
        out_shape=jax.ShapeDtypeStruct((N_PTS, 2 * CH), jnp.float32),
    )(features, W1, gamma1.reshape(1, CH), beta1.reshape(1, CH),
      W2, gamma2.reshape(1, CH), beta2.reshape(1, CH), counts_col)


def _sc_gather_max(c_rows, idx_flat):
    num_cores, num_subcores = 2, 16
    nw = num_cores * num_subcores
    per_w = N_SMP * KNN // nw
    chunk = 128
    n_chunks = per_w // chunk
    smp_per_chunk = chunk // KNN
    mesh = plsc.VectorSubcoreMesh(core_axis_name="c", subcore_axis_name="s",
                                  num_cores=num_cores,
                                  num_subcores=num_subcores)

    width = 2 * CH

    @functools.partial(
        pl.kernel,
        out_type=jax.ShapeDtypeStruct((N_SMP, width), jnp.float32),
        mesh=mesh,
        scratch_types=[
            pltpu.VMEM((chunk,), jnp.int32),
            pltpu.VMEM((chunk, width), jnp.float32),
            pltpu.VMEM((smp_per_chunk, width), jnp.float32),
            pltpu.SemaphoreType.DMA,
        ],
    )
    def run(c_hbm, idx_hbm, out_hbm, idx_v, rows_v, res_v, sem):
        wid = lax.axis_index("s") * num_cores + lax.axis_index("c")
        base = wid * per_w

        def do_chunk(ch, carry):
            pltpu.sync_copy(idx_hbm.at[pl.ds(base + ch * chunk, chunk)],
                            idx_v)
            pltpu.async_copy(c_hbm.at[idx_v], rows_v, sem).wait()
            for s in range(smp_per_chunk):
                for l in range(width // 16):
                    acc = rows_v[s * KNN, pl.ds(l * 16, 16)]
                    for j in range(1, KNN):
                        acc = jnp.maximum(
                            acc, rows_v[s * KNN + j, pl.ds(l * 16, 16)])
                    res_v[s, pl.ds(l * 16, 16)] = acc
            row0 = wid * (per_w // KNN) + ch * smp_per_chunk
            pltpu.sync_copy(res_v,
                            out_hbm.at[pl.ds(row0, smp_per_chunk)])
            return carry

        lax.fori_loop(0, n_chunks, do_chunk, jnp.int32(0))

    return run(c_rows, idx_flat)


def kernel(points, features, W1, gamma1, beta1, W2, gamma2, beta2):
    sampled_points = _fps(points)
    knn_idx, counts = _knn(sampled_points, points.T)
    counts_col = counts.reshape(N_PTS, 1)
    c_rows = _mlp(features, W1, gamma1, beta1, W2, gamma2, beta2, counts_col)
    out_features = _sc_gather_max(c_rows, knn_idx.reshape(-1))[:, :CH]
    return (sampled_points, out_features)
